# Initial kernel scaffold; baseline (speedup 1.0000x reference)
import functools

import jax
import jax.numpy as jnp
from jax import lax
from jax.experimental import pallas as pl
from jax.experimental.pallas import tpu as pltpu
from jax.experimental.pallas import tpu_sc as plsc

D = 128
E = 320000
NC, NS, L = 2, 16, 16
NW = NC * NS
EPW = E // NW
C = 400
NCHUNK = EPW // C
SUB = 80
NSUB = C // SUB
G = C // L

_mesh = plsc.VectorSubcoreMesh(core_axis_name="c", subcore_axis_name="s")


@functools.partial(
    pl.kernel,
    out_type=jax.ShapeDtypeStruct((E,), jnp.float32),
    mesh=_mesh,
    scratch_types=[
        pltpu.VMEM((C,), jnp.int32),
        pltpu.VMEM((C,), jnp.int32),
        pltpu.VMEM((C, D), jnp.float32),
        pltpu.VMEM((C, D), jnp.float32),
        pltpu.VMEM((C,), jnp.float32),
        pltpu.SemaphoreType.DMA,
    ],
)
def _ip_decode(xu_hbm, xi_hbm, src_hbm, dst_hbm, out_hbm,
               src_v, dst_v, u_v, v_v, o_v, sem):
    wid = lax.axis_index("s") * NC + lax.axis_index("c")
    base = wid * EPW

    def chunk_body(ci, carry):
        off = base + ci * C
        pltpu.sync_copy(src_hbm.at[pl.ds(off, C)], src_v)
        pltpu.sync_copy(dst_hbm.at[pl.ds(off, C)], dst_v)
        copies = []
        for j in range(NSUB):
            sl = pl.ds(j * SUB, SUB)
            copies.append(pltpu.async_copy(xu_hbm.at[src_v.at[sl]], u_v.at[sl], sem))
            copies.append(pltpu.async_copy(xi_hbm.at[dst_v.at[sl]], v_v.at[sl], sem))
        for cp in copies:
            cp.wait()

        def group_body(g, carry2):
            eids = g * L + lax.iota(jnp.int32, L)

            def d_body(d, acc):
                dcol = jnp.full((L,), d, dtype=jnp.int32)
                du = plsc.load_gather(u_v, [eids, dcol])
                dv = plsc.load_gather(v_v, [eids, dcol])
                return acc + du * dv

            acc = lax.fori_loop(0, D, d_body, jnp.zeros((L,), jnp.float32))
            o_v[pl.ds(g * L, L)] = acc
            return carry2

        lax.fori_loop(0, G, group_body, 0)
        pltpu.sync_copy(o_v, out_hbm.at[pl.ds(off, C)])
        return carry

    lax.fori_loop(0, NCHUNK, chunk_body, 0)


def kernel(x_user, x_item, edge_label_index):
    eli = edge_label_index.astype(jnp.int32)
    return _ip_decode(x_user, x_item, eli[0], eli[1])


# speedup vs baseline: 1.2086x; 1.2086x over previous
import functools

import jax
import jax.numpy as jnp
from jax import lax
from jax.experimental import pallas as pl
from jax.experimental.pallas import tpu as pltpu
from jax.experimental.pallas import tpu_sc as plsc

D = 128
E = 320000
NC, NS, L = 2, 16, 16
NW = NC * NS
EPW = E // NW
C = 400
NCHUNK = EPW // C
SUB = 80
NSUB = C // SUB
G = C // L

_mesh = plsc.VectorSubcoreMesh(core_axis_name="c", subcore_axis_name="s")


@functools.partial(
    pl.kernel,
    out_type=jax.ShapeDtypeStruct((E,), jnp.float32),
    mesh=_mesh,
    scratch_types=[
        pltpu.VMEM((C,), jnp.int32),
        pltpu.VMEM((C,), jnp.int32),
        pltpu.VMEM((C, D), jnp.float32),
        pltpu.VMEM((C, D), jnp.float32),
        pltpu.VMEM((C,), jnp.float32),
        pltpu.SemaphoreType.DMA,
    ],
    compiler_params=pltpu.CompilerParams(needs_layout_passes=False),
)
def _ip_decode(xu_hbm, xi_hbm, src_hbm, dst_hbm, out_hbm,
               src_v, dst_v, u_v, v_v, o_v, sem):
    wid = lax.axis_index("s") * NC + lax.axis_index("c")
    base = wid * EPW

    def chunk_body(ci, carry):
        off = base + ci * C
        pltpu.sync_copy(src_hbm.at[pl.ds(off, C)], src_v)
        pltpu.sync_copy(dst_hbm.at[pl.ds(off, C)], dst_v)
        copies = []
        for j in range(NSUB):
            sl = pl.ds(j * SUB, SUB)
            copies.append(pltpu.async_copy(xu_hbm.at[src_v.at[sl]], u_v.at[sl], sem))
            copies.append(pltpu.async_copy(xi_hbm.at[dst_v.at[sl]], v_v.at[sl], sem))
        for cp in copies:
            cp.wait()

        def group_body(g, carry2):
            eids = g * L + lax.iota(jnp.int32, L)

            def d_body(d, acc):
                dcol = jnp.full((L,), d, dtype=jnp.int32)
                du = plsc.load_gather(u_v, [eids, dcol])
                dv = plsc.load_gather(v_v, [eids, dcol])
                return acc + du * dv

            acc = lax.fori_loop(0, D, d_body, jnp.zeros((L,), jnp.float32))
            o_v[pl.ds(g * L, L)] = acc
            return carry2

        lax.fori_loop(0, G, group_body, 0)
        pltpu.sync_copy(o_v, out_hbm.at[pl.ds(off, C)])
        return carry

    lax.fori_loop(0, NCHUNK, chunk_body, 0)


def kernel(x_user, x_item, edge_label_index):
    eli = edge_label_index.astype(jnp.int32)
    return _ip_decode(x_user, x_item, eli[0], eli[1])


# speedup vs baseline: 1.2135x; 1.0041x over previous
import functools

import jax
import jax.numpy as jnp
from jax import lax
from jax.experimental import pallas as pl
from jax.experimental.pallas import tpu as pltpu
from jax.experimental.pallas import tpu_sc as plsc

D = 128
E = 320000
NC, NS, L = 2, 16, 16
NW = NC * NS
EPW = E // NW
C = 400
NCHUNK = EPW // C
SUB = 80
NSUB = C // SUB
G = C // L

_mesh = plsc.VectorSubcoreMesh(core_axis_name="c", subcore_axis_name="s")


@functools.partial(
    pl.kernel,
    out_type=jax.ShapeDtypeStruct((E,), jnp.float32),
    mesh=_mesh,
    scratch_types=[
        pltpu.VMEM((C,), jnp.int32),
        pltpu.VMEM((C,), jnp.int32),
        pltpu.VMEM((C, D), jnp.float32),
        pltpu.VMEM((C, D), jnp.float32),
        pltpu.VMEM((C,), jnp.float32),
        pltpu.SemaphoreType.DMA,
    ],
    compiler_params=pltpu.CompilerParams(needs_layout_passes=False),
)
def _ip_decode(xu_hbm, xi_hbm, src_hbm, dst_hbm, out_hbm,
               src_v, dst_v, u_v, v_v, o_v, sem):
    wid = lax.axis_index("s") * NC + lax.axis_index("c")
    base = wid * EPW

    def chunk_body(ci, carry):
        off = base + ci * C
        pltpu.sync_copy(src_hbm.at[pl.ds(off, C)], src_v)
        pltpu.sync_copy(dst_hbm.at[pl.ds(off, C)], dst_v)
        copies = []
        for j in range(NSUB):
            sl = pl.ds(j * SUB, SUB)
            copies.append(pltpu.async_copy(xu_hbm.at[src_v.at[sl]], u_v.at[sl], sem))
            copies.append(pltpu.async_copy(xi_hbm.at[dst_v.at[sl]], v_v.at[sl], sem))
        for cp in copies:
            cp.wait()

        def group_body(g, carry2):
            eids = g * L + lax.iota(jnp.int32, L)
            acc = jnp.zeros((L,), jnp.float32)
            for d in range(D):
                dcol = jnp.full((L,), d, dtype=jnp.int32)
                du = plsc.load_gather(u_v, [eids, dcol])
                dv = plsc.load_gather(v_v, [eids, dcol])
                acc = acc + du * dv
            o_v[pl.ds(g * L, L)] = acc
            return carry2

        lax.fori_loop(0, G, group_body, 0)
        pltpu.sync_copy(o_v, out_hbm.at[pl.ds(off, C)])
        return carry

    lax.fori_loop(0, NCHUNK, chunk_body, 0)


def kernel(x_user, x_item, edge_label_index):
    eli = edge_label_index.astype(jnp.int32)
    return _ip_decode(x_user, x_item, eli[0], eli[1])


# speedup vs baseline: 2.7170x; 2.2390x over previous
import functools

import jax
import jax.numpy as jnp
from jax import lax
from jax.experimental import pallas as pl
from jax.experimental.pallas import tpu as pltpu
from jax.experimental.pallas import tpu_sc as plsc

D = 128
E = 320000
NV = 10000
NC, NS, L = 2, 16, 16
NW = NC * NS
FPW = D // NW
SLICE = NV * FPW
C = 2000
NCH = E // C
GG = C // (5 * L)

_mesh = plsc.VectorSubcoreMesh(core_axis_name="c", subcore_axis_name="s")


@functools.partial(
    pl.kernel,
    out_type=jax.ShapeDtypeStruct((NW * E,), jnp.float32),
    mesh=_mesh,
    scratch_types=[
        pltpu.VMEM((SLICE,), jnp.float32),
        pltpu.VMEM((SLICE,), jnp.float32),
        pltpu.VMEM((C,), jnp.int32),
        pltpu.VMEM((C,), jnp.int32),
        pltpu.VMEM((C,), jnp.int32),
        pltpu.VMEM((C,), jnp.int32),
        pltpu.VMEM((C,), jnp.float32),
        pltpu.VMEM((C,), jnp.float32),
        pltpu.SemaphoreType.DMA,
        pltpu.SemaphoreType.DMA,
        pltpu.SemaphoreType.DMA,
        pltpu.SemaphoreType.DMA,
    ],
    compiler_params=pltpu.CompilerParams(needs_layout_passes=False),
)
def _partial_dots(xu_hbm, xi_hbm, src_hbm, dst_hbm, part_hbm,
                  u_sl, v_sl, sv0, sv1, dv0, dv1, ov0, ov1,
                  qi0, qi1, qo0, qo1):
    wid = lax.axis_index("s") * NC + lax.axis_index("c")
    pltpu.sync_copy(xu_hbm.at[wid], u_sl)
    pltpu.sync_copy(xi_hbm.at[wid], v_sl)

    svs, dvs, ovs = (sv0, sv1), (dv0, dv1), (ov0, ov1)
    qis, qos = (qi0, qi1), (qo0, qo1)

    def fire_idx(ci, b):
        off = ci * C
        pltpu.async_copy(src_hbm.at[pl.ds(off, C)], svs[b], qis[b])
        pltpu.async_copy(dst_hbm.at[pl.ds(off, C)], dvs[b], qis[b])

    def drain_idx(b):
        pltpu.make_async_copy(src_hbm.at[pl.ds(0, C)], svs[b], qis[b]).wait()
        pltpu.make_async_copy(dst_hbm.at[pl.ds(0, C)], dvs[b], qis[b]).wait()

    def fire_out(ci, b):
        off = pl.multiple_of(wid * E + ci * C, 8)
        pltpu.async_copy(ovs[b], part_hbm.at[pl.ds(off, C)], qos[b])

    def drain_out(b):
        pltpu.make_async_copy(
            ovs[b], part_hbm.at[pl.ds(0, C)], qos[b]).wait()

    def compute(ci, b):
        drain_idx(b)

        def gg_body(gg, carry):
            for j in range(5):
                off16 = gg * (5 * L) + j * L
                s16 = svs[b][pl.ds(off16, L)] * FPW
                d16 = dvs[b][pl.ds(off16, L)] * FPW
                acc = jnp.zeros((L,), jnp.float32)
                for f in range(FPW):
                    uf = plsc.load_gather(u_sl, [s16 + f])
                    vf = plsc.load_gather(v_sl, [d16 + f])
                    acc = acc + uf * vf
                ovs[b][pl.ds(off16, L)] = acc
            return carry

        lax.fori_loop(0, GG, gg_body, 0)

    fire_idx(0, 0)

    def step(k, carry):
        i0 = 2 * k
        i1 = i0 + 1
        fire_idx(i1, 1)

        @pl.when(k > 0)
        def _():
            drain_out(0)

        compute(i0, 0)
        fire_out(i0, 0)

        @pl.when(k < NCH // 2 - 1)
        def _():
            fire_idx(i1 + 1, 0)

        @pl.when(k > 0)
        def _():
            drain_out(1)

        compute(i1, 1)
        fire_out(i1, 1)
        return carry

    lax.fori_loop(0, NCH // 2, step, 0)
    drain_out(0)
    drain_out(1)


BK = 1280
NB = E // BK


def _rowsum_body(p_ref, o_ref):
    o_ref[0, 0, :] = jnp.sum(p_ref[...], axis=0)


_rowsum = pl.pallas_call(
    _rowsum_body,
    out_shape=jax.ShapeDtypeStruct((NB, 1, BK), jnp.float32),
    grid=(NB,),
    in_specs=[pl.BlockSpec((NW, BK), lambda i: (0, i))],
    out_specs=pl.BlockSpec((1, 1, BK), lambda i: (i, 0, 0)),
)


def kernel(x_user, x_item, edge_label_index):
    eli = edge_label_index.astype(jnp.int32)
    xu_cols = x_user.reshape(NV, NW, FPW).transpose(1, 0, 2).reshape(NW, SLICE)
    xi_cols = x_item.reshape(NV, NW, FPW).transpose(1, 0, 2).reshape(NW, SLICE)
    part = _partial_dots(xu_cols, xi_cols, eli[0], eli[1])
    return _rowsum(part.reshape(NW, E)).reshape(E)


# speedup vs baseline: 5.1720x; 1.9036x over previous
import functools

import jax
import jax.numpy as jnp
from jax import lax
from jax.experimental import pallas as pl
from jax.experimental.pallas import tpu as pltpu
from jax.experimental.pallas import tpu_sc as plsc

D = 128
E = 320000
NV = 10000
NC, NS, L = 2, 16, 16
NWRD = D // 2
WPS = NWRD // NS
E2 = E // NC
C = 2000
NCH = E2 // C
GG = C // (5 * L)

_mesh = plsc.VectorSubcoreMesh(core_axis_name="c", subcore_axis_name="s")


@functools.partial(
    pl.kernel,
    out_type=jax.ShapeDtypeStruct((NS * E,), jnp.float32),
    mesh=_mesh,
    scratch_types=[
        pltpu.VMEM((NV,), jnp.int32),
        pltpu.VMEM((NV,), jnp.int32),
        pltpu.VMEM((NV,), jnp.int32),
        pltpu.VMEM((NV,), jnp.int32),
        pltpu.VMEM((NV,), jnp.int32),
        pltpu.VMEM((NV,), jnp.int32),
        pltpu.VMEM((NV,), jnp.int32),
        pltpu.VMEM((NV,), jnp.int32),
        pltpu.VMEM((C,), jnp.int32),
        pltpu.VMEM((C,), jnp.int32),
        pltpu.VMEM((C,), jnp.int32),
        pltpu.VMEM((C,), jnp.int32),
        pltpu.VMEM((C,), jnp.float32),
        pltpu.VMEM((C,), jnp.float32),
        pltpu.SemaphoreType.DMA,
        pltpu.SemaphoreType.DMA,
        pltpu.SemaphoreType.DMA,
        pltpu.SemaphoreType.DMA,
    ],
    compiler_params=pltpu.CompilerParams(needs_layout_passes=False),
)
def _partial_dots(xu_hbm, xi_hbm, src_hbm, dst_hbm, part_hbm,
                  uw0, uw1, uw2, uw3, vw0, vw1, vw2, vw3,
                  sv0, sv1, dv0, dv1, ov0, ov1,
                  qi0, qi1, qo0, qo1):
    cid = lax.axis_index("c")
    sid = lax.axis_index("s")
    ebase = cid * E2

    u_w = (uw0, uw1, uw2, uw3)
    v_w = (vw0, vw1, vw2, vw3)
    for k in range(WPS):
        pltpu.sync_copy(xu_hbm.at[sid * WPS + k], u_w[k])
        pltpu.sync_copy(xi_hbm.at[sid * WPS + k], v_w[k])

    svs, dvs, ovs = (sv0, sv1), (dv0, dv1), (ov0, ov1)
    qis, qos = (qi0, qi1), (qo0, qo1)

    def fire_idx(ci, b):
        off = pl.multiple_of(ebase + ci * C, 8)
        pltpu.async_copy(src_hbm.at[pl.ds(off, C)], svs[b], qis[b])
        pltpu.async_copy(dst_hbm.at[pl.ds(off, C)], dvs[b], qis[b])

    def drain_idx(b):
        pltpu.make_async_copy(src_hbm.at[pl.ds(0, C)], svs[b], qis[b]).wait()
        pltpu.make_async_copy(dst_hbm.at[pl.ds(0, C)], dvs[b], qis[b]).wait()

    def fire_out(ci, b):
        off = pl.multiple_of(sid * E + ebase + ci * C, 8)
        pltpu.async_copy(ovs[b], part_hbm.at[pl.ds(off, C)], qos[b])

    def drain_out(b):
        pltpu.make_async_copy(
            ovs[b], part_hbm.at[pl.ds(0, C)], qos[b]).wait()

    def compute(ci, b):
        drain_idx(b)

        @plsc.parallel_loop(0, C // L, unroll=4)
        def _grp(g):
            off16 = g * L
            s16 = svs[b][pl.ds(off16, L)]
            d16 = dvs[b][pl.ds(off16, L)]
            acc_e = jnp.zeros((L,), jnp.float32)
            acc_o = jnp.zeros((L,), jnp.float32)
            for k in range(WPS):
                uw = plsc.load_gather(u_w[k], [s16])
                vw = plsc.load_gather(v_w[k], [d16])
                ub = plsc.bitcast(uw, jnp.bfloat16)
                vb = plsc.bitcast(vw, jnp.bfloat16)
                pe, po = plsc.unpack(
                    ub * vb, format=plsc.PackFormat.INTERLEAVED)
                acc_e = acc_e + pe
                acc_o = acc_o + po
            ovs[b][pl.ds(off16, L)] = acc_e + acc_o

    fire_idx(0, 0)

    def step(k, carry):
        i0 = 2 * k
        i1 = i0 + 1
        fire_idx(i1, 1)

        @pl.when(k > 0)
        def _():
            drain_out(0)

        compute(i0, 0)
        fire_out(i0, 0)

        @pl.when(k < NCH // 2 - 1)
        def _():
            fire_idx(i1 + 1, 0)

        @pl.when(k > 0)
        def _():
            drain_out(1)

        compute(i1, 1)
        fire_out(i1, 1)
        return carry

    lax.fori_loop(0, NCH // 2, step, 0)
    drain_out(0)
    drain_out(1)


BK = 1280
NB = E // BK


def _rowsum_body(p_ref, o_ref):
    o_ref[0, 0, :] = jnp.sum(p_ref[...], axis=0)


_rowsum = pl.pallas_call(
    _rowsum_body,
    out_shape=jax.ShapeDtypeStruct((NB, 1, BK), jnp.float32),
    grid=(NB,),
    in_specs=[pl.BlockSpec((NS, BK), lambda i: (0, i))],
    out_specs=pl.BlockSpec((1, 1, BK), lambda i: (i, 0, 0)),
)


def _pack_words(x):
    xb = x.astype(jnp.bfloat16).reshape(NV, NWRD, 2)
    words = jax.lax.bitcast_convert_type(xb, jnp.int32)
    return words.T.reshape(NWRD, NV)


def kernel(x_user, x_item, edge_label_index):
    eli = edge_label_index.astype(jnp.int32)
    part = _partial_dots(_pack_words(x_user), _pack_words(x_item),
                         eli[0], eli[1])
    return _rowsum(part.reshape(NS, E)).reshape(E)
